# baseline (speedup 1.0000x reference)
import functools

import jax
import jax.numpy as jnp
from jax import lax
from jax.experimental import pallas as pl
from jax.experimental.pallas import tpu as pltpu
from jax.experimental.pallas import tpu_sc as plsc

NUM_CORES = 2
NUM_SUBCORES = 16
NUM_WORKERS = NUM_CORES * NUM_SUBCORES
LANES = 16
IDX_CHUNK = 100


def _build_sc_kernel(B, S, E, V):
    R = B * S
    rows_per_worker = R // NUM_WORKERS
    steps = rows_per_worker // S
    pairs_per_step = S // IDX_CHUNK

    mesh = plsc.VectorSubcoreMesh(
        core_axis_name="c", subcore_axis_name="s",
        num_cores=NUM_CORES, num_subcores=NUM_SUBCORES)

    @functools.partial(
        pl.kernel,
        out_type=jax.ShapeDtypeStruct((R, E), jnp.float32),
        mesh=mesh,
        scratch_types=[
            pltpu.VMEM((pairs_per_step, IDX_CHUNK), jnp.int32),
            pltpu.VMEM((S, E), jnp.float32),
            pltpu.VMEM((S, E), jnp.float32),
            pltpu.SemaphoreType.DMA,
        ],
        compiler_params=pltpu.CompilerParams(use_tc_tiling_on_sc=False),
    )
    def k(x2, table, pos_hbm, out, idx_v, rows_v, pos_v, sem):
        c = lax.axis_index("c")
        s = lax.axis_index("s")
        wid = s * NUM_CORES + c
        pltpu.sync_copy(pos_hbm, pos_v)
        base_pair = wid * (rows_per_worker // IDX_CHUNK)

        @pl.loop(0, steps)
        def _step(step):
            pair0 = base_pair + step * pairs_per_step
            pltpu.sync_copy(x2.at[pl.ds(pair0, pairs_per_step)], idx_v)
            cps = [
                pltpu.async_copy(
                    table.at[idx_v.at[p]],
                    rows_v.at[pl.ds(p * IDX_CHUNK, IDX_CHUNK)],
                    sem)
                for p in range(pairs_per_step)
            ]
            for cp in cps:
                cp.wait()

            @pl.loop(0, S)
            def _row(r):
                for j in range(E // LANES):
                    sl = pl.ds(j * LANES, LANES)
                    rows_v[r, sl] = rows_v[r, sl] + pos_v[r, sl]

            row0 = wid * rows_per_worker + step * S
            pltpu.sync_copy(rows_v, out.at[pl.ds(row0, S)])

    return k


def kernel(x, table, pos_encoding):
    B, S = x.shape
    V, E = table.shape
    x2 = x.astype(jnp.int32).reshape(-1, IDX_CHUNK)
    pos_s = pos_encoding[0, :S, :]
    out = _build_sc_kernel(B, S, E, V)(x2, table, pos_s)
    return out.reshape(B, S, E)
